# baseline (speedup 1.0000x reference)
import functools

import jax
import jax.numpy as jnp
from jax import lax
from jax.experimental import pallas as pl
from jax.experimental.pallas import tpu as pltpu
from jax.experimental.pallas import tpu_sc as plsc

_N = 5000
_M = 100
_NP = 5120
_NT = 16
_PT = _NP // _NT
_PC = _PT // 16
_BIGI = 1 << 30
_LBL = 128
_LC = 7


def _lanesum16(x):
    s = x[0]
    for i in range(1, 16):
        s = s + x[i]
    return s


def _sc_body(scores_h, segf_h, gtf_h, out_h,
             seg_v, gt_v, amin_v, amax_v, conf_v, bmin_v, bmax_v,
             first_v, firstall_v, taken_v, chosen_v, chosenb_v, cc_v,
             rpart_v, partall_v, rank_v, accp_v, tbuf_v, res_v,
             sh_first, sh_chosen, sh_part):
    w = lax.axis_index("s")
    iota16 = lax.iota(jnp.int32, 16)
    lane0 = iota16 == 0

    for q in range(2 * _N // 16, 2 * _NP // 16):
        seg_v[pl.ds(q * 16, 16)] = jnp.full((16,), -1.0e6, jnp.float32)
    for q in range(200 // 16, 256 // 16):
        gt_v[pl.ds(q * 16, 16)] = jnp.full((16,), 2.0e6, jnp.float32)
    for q in range(_N // 16 - 1, _NP // 16):
        conf_v[pl.ds(q * 16, 16)] = jnp.full((16,), -1.0, jnp.float32)
    pltpu.sync_copy(segf_h, seg_v.at[pl.ds(0, 2 * _N)])
    pltpu.sync_copy(gtf_h, gt_v.at[pl.ds(0, 200)])
    pltpu.sync_copy(scores_h, conf_v.at[pl.ds(0, _N)])

    for q in range(_NP // 16):
        even = iota16 * 2 + q * 32
        amin_v[pl.ds(q * 16, 16)] = plsc.load_gather(seg_v, [even])
        amax_v[pl.ds(q * 16, 16)] = plsc.load_gather(seg_v, [even + 1])
    for q in range(_LC):
        even = jnp.minimum(iota16 * 2 + q * 32, 254)
        bmin_v[pl.ds(q * 16, 16)] = plsc.load_gather(gt_v, [even])
        bmax_v[pl.ds(q * 16, 16)] = plsc.load_gather(gt_v, [even + 1])
    fix = jnp.where(iota16 >= 4, 2.0e6, bmin_v[pl.ds(96, 16)])
    bmin_v[pl.ds(96, 16)] = fix
    fix2 = jnp.where(iota16 >= 4, 2.0e6, bmax_v[pl.ds(96, 16)])
    bmax_v[pl.ds(96, 16)] = fix2

    base = w * _PT

    def phase_a_label(i, jb):
        jidx = jnp.full((16,), jb + i, jnp.int32)
        b0 = plsc.load_gather(bmin_v, [jidx])
        b1 = plsc.load_gather(bmax_v, [jidx])
        blen = b1 - b0
        acc = jnp.full((16,), _BIGI, jnp.int32)
        for q in range(_PC):
            a0 = amin_v[pl.ds(base + q * 16, 16)]
            a1 = amax_v[pl.ds(base + q * 16, 16)]
            inter = jnp.maximum(jnp.minimum(a1, b1) - jnp.maximum(a0, b0), 0.0)
            union = (a1 - a0) + blen - inter
            cand = inter >= 0.5 * union - 1.0e-3
            idx = iota16 + (base + q * 16)
            acc = jnp.minimum(acc, jnp.where(cand, idx, _BIGI))
        tbuf_v[pl.ds(i * 16, 16)] = acc
        return jb

    def phase_a_chunk(jc, _):
        lax.fori_loop(0, 16, phase_a_label, jc * 16)
        res = jnp.full((16,), _BIGI, jnp.int32)
        for c in range(16):
            col = plsc.load_gather(tbuf_v, [iota16 * 16 + c])
            res = jnp.minimum(res, col)
        first_v[pl.ds(jc * 16, 16)] = res
        return 0

    lax.fori_loop(0, _LC, phase_a_chunk, 0)
    pltpu.sync_copy(first_v, sh_first.at[pl.ds(w * _LBL, _LBL)])
    plsc.subcore_barrier()

    @pl.when(w == 0)
    def _phase_b():
        for q in range(_NP // 16):
            taken_v[pl.ds(q * 16, 16)] = jnp.zeros((16,), jnp.int32)
        pltpu.sync_copy(sh_first, firstall_v)

        def red_first(j, _):
            acc = jnp.full((16,), _BIGI, jnp.int32)
            for t in range(_NT):
                acc = jnp.minimum(
                    acc, firstall_v[pl.ds(t * _LBL + j * 16, 16)])
            chosenb_v[pl.ds(j * 16, 16)] = acc
            return 0
        lax.fori_loop(0, _LC, red_first, 0)

        def phase_b_label(j, _):
            jidx = jnp.full((16,), j, jnp.int32)
            first = plsc.load_gather(chosenb_v, [jidx])[0]
            c0 = jnp.where(first < _BIGI,
                           lax.shift_right_logical(first, 4), 10 ** 6)
            b0 = plsc.load_gather(bmin_v, [jidx])
            b1 = plsc.load_gather(bmax_v, [jidx])
            blen = b1 - b0

            def cond(st):
                c, chosen = st
                return (c < _NP // 16) & (chosen >= _BIGI)

            def step(st):
                c, _ = st
                a0 = amin_v[pl.ds(c * 16, 16)]
                a1 = amax_v[pl.ds(c * 16, 16)]
                inter = jnp.maximum(
                    jnp.minimum(a1, b1) - jnp.maximum(a0, b0), 0.0)
                union = (a1 - a0) + blen - inter
                iou = inter / union
                tak = taken_v[pl.ds(c * 16, 16)]
                free = (iou > 0.5) & (tak == 0)
                fv = plsc.all_reduce_ffs(free)[0]
                ch = jnp.where(fv < 16, c * 16 + fv, jnp.int32(_BIGI))
                return c + 1, ch

            _, chosen = lax.while_loop(
                cond, step, (jnp.minimum(c0, _NP // 16), jnp.int32(_BIGI)))
            has = chosen < _BIGI
            one = jnp.full((16,), 1, jnp.int32)
            plsc.store_scatter(
                taken_v,
                [jnp.full((16,), jnp.minimum(chosen, _NP - 1), jnp.int32)],
                one, mask=lane0 & has)
            plsc.store_scatter(chosen_v, [jidx],
                               jnp.full((16,), chosen, jnp.int32), mask=lane0)
            return 0

        for q in range(_LBL // 16):
            chosen_v[pl.ds(q * 16, 16)] = jnp.full((16,), _BIGI, jnp.int32)
        lax.fori_loop(0, _M, phase_b_label, 0)
        pltpu.sync_copy(chosen_v, sh_chosen)

    plsc.subcore_barrier()

    pltpu.sync_copy(sh_chosen, chosenb_v)
    for q in range(_LBL // 16):
        idx = chosenb_v[pl.ds(q * 16, 16)]
        vmask = idx < _BIGI
        cidx = jnp.minimum(idx, _N - 1)
        cc = plsc.load_gather(conf_v, [cidx], mask=vmask)
        cc_v[pl.ds(q * 16, 16)] = jnp.where(vmask, cc, -9.0)

    def phase_c_label(i, kb):
        kidx = jnp.full((16,), kb + i, jnp.int32)
        cvec = plsc.load_gather(cc_v, [kidx])
        mivec = plsc.load_gather(chosenb_v, [kidx])
        acc = jnp.zeros((16,), jnp.int32)
        for q in range(_PC):
            cf = conf_v[pl.ds(base + q * 16, 16)]
            gi = iota16 + (base + q * 16)
            acc = acc + jnp.where(cf > cvec, 1, 0)
            acc = acc + jnp.where((cf == cvec) & (gi < mivec), 1, 0)
        tbuf_v[pl.ds(i * 16, 16)] = acc
        return kb

    def phase_c_chunk(kc, _):
        lax.fori_loop(0, 16, phase_c_label, kc * 16)
        res = jnp.zeros((16,), jnp.int32)
        for c in range(16):
            res = res + plsc.load_gather(tbuf_v, [iota16 * 16 + c])
        rpart_v[pl.ds(kc * 16, 16)] = res.astype(jnp.float32)
        return 0

    lax.fori_loop(0, _LC, phase_c_chunk, 0)
    pltpu.sync_copy(rpart_v, sh_part.at[pl.ds(w * _LBL, _LBL)])
    plsc.subcore_barrier()

    @pl.when(w == 0)
    def _phase_d():
        pltpu.sync_copy(sh_part, partall_v)

        def red_part(q, _):
            acc = jnp.zeros((16,), jnp.float32)
            for t in range(_NT):
                acc = acc + partall_v[pl.ds(t * _LBL + q * 16, 16)]
            rank_v[pl.ds(q * 16, 16)] = acc
            return 0
        lax.fori_loop(0, _LC, red_part, 0)

        def acc_loop(l, _):
            lidx = jnp.full((16,), l, jnp.int32)
            r_l = plsc.load_gather(rank_v, [lidx])
            v_l = jnp.where(
                plsc.load_gather(chosenb_v, [lidx]) < _BIGI, 1.0, 0.0)
            for q in range(_LC):
                rk = rank_v[pl.ds(q * 16, 16)]
                upd = jnp.where(r_l <= rk, v_l, 0.0)
                accp_v[pl.ds(q * 16, 16)] = accp_v[pl.ds(q * 16, 16)] + upd
            return 0

        for q in range(_LC):
            accp_v[pl.ds(q * 16, 16)] = jnp.zeros((16,), jnp.float32)
        lax.fori_loop(0, _M, acc_loop, 0)

        for q in range(_LC):
            rk = rank_v[pl.ds(q * 16, 16)]
            ak = accp_v[pl.ds(q * 16, 16)]
            accp_v[pl.ds(q * 16, 16)] = ak / (rk + 1.0)

        def suff_loop(l, suf):
            lidx = jnp.full((16,), l, jnp.int32)
            r_l = plsc.load_gather(rank_v, [lidx])
            p_l = plsc.load_gather(accp_v, [lidx])
            v_l = plsc.load_gather(chosenb_v, [lidx]) < _BIGI
            out = []
            for q in range(_LC):
                rk = rank_v[pl.ds(q * 16, 16)]
                s = jnp.where(v_l & (r_l >= rk), p_l, 0.0)
                out.append(jnp.maximum(suf[q], s))
            return tuple(out)

        suf0 = tuple(jnp.zeros((16,), jnp.float32) for _ in range(_LC))
        suf = lax.fori_loop(0, _M, suff_loop, suf0)

        apv = jnp.zeros((16,), jnp.float32)
        for q in range(_LC):
            rk = rank_v[pl.ds(q * 16, 16)]
            vk = chosenb_v[pl.ds(q * 16, 16)] < _BIGI
            apv = apv + jnp.where(vk & (rk >= 1.0), suf[q], 0.0)
        ap = _lanesum16(apv)
        res_v[...] = jnp.full((16,), ap * (1.0 / _M), jnp.float32)
        pltpu.sync_copy(res_v, out_h)


@jax.jit
def kernel(scores, segments, gt):
    segf = segments.reshape(2 * _N)
    gtf = gt.reshape(2 * _M)

    mesh = plsc.VectorSubcoreMesh(
        core_axis_name="c", subcore_axis_name="s", num_cores=1)
    f = functools.partial(
        pl.kernel, mesh=mesh,
        out_type=jax.ShapeDtypeStruct((16,), jnp.float32),
        compiler_params=pltpu.CompilerParams(needs_layout_passes=False),
        scratch_types=[
            pltpu.VMEM((2 * _NP,), jnp.float32),
            pltpu.VMEM((256,), jnp.float32),
            pltpu.VMEM((_NP,), jnp.float32),
            pltpu.VMEM((_NP,), jnp.float32),
            pltpu.VMEM((_NP,), jnp.float32),
            pltpu.VMEM((_LBL,), jnp.float32),
            pltpu.VMEM((_LBL,), jnp.float32),
            pltpu.VMEM((_LBL,), jnp.int32),
            pltpu.VMEM((_NT * _LBL,), jnp.int32),
            pltpu.VMEM((_NP,), jnp.int32),
            pltpu.VMEM((_LBL,), jnp.int32),
            pltpu.VMEM((_LBL,), jnp.int32),
            pltpu.VMEM((_LBL,), jnp.float32),
            pltpu.VMEM((_LBL,), jnp.float32),
            pltpu.VMEM((_NT * _LBL,), jnp.float32),
            pltpu.VMEM((_LBL,), jnp.float32),
            pltpu.VMEM((_LBL,), jnp.float32),
            pltpu.VMEM((256,), jnp.int32),
            pltpu.VMEM((16,), jnp.float32),
            pltpu.VMEM_SHARED((_NT * _LBL,), jnp.int32),
            pltpu.VMEM_SHARED((_LBL,), jnp.int32),
            pltpu.VMEM_SHARED((_NT * _LBL,), jnp.float32),
        ])(_sc_body)
    out = f(scores, segf, gtf)
    return out[0]


# speedup vs baseline: 1.1046x; 1.1046x over previous
import functools

import jax
import jax.numpy as jnp
from jax import lax
from jax.experimental import pallas as pl
from jax.experimental.pallas import tpu as pltpu
from jax.experimental.pallas import tpu_sc as plsc

_N = 5000
_M = 100
_NP = 5120
_NT = 16
_PT = _NP // _NT
_PC = _PT // 16
_BIGI = 1 << 30
_LBL = 128
_LC = 7


def _lanesum16(x):
    s = x[0]
    for i in range(1, 16):
        s = s + x[i]
    return s


def _sc_body(scores_h, segf_h, gtf_h, out_h,
             seg_v, gt_v, amin_v, amax_v, conf_v, bmin_v, bmax_v,
             first_v, firstall_v, taken_v, chosen_v, chosenb_v, cc_v,
             rpart_v, partall_v, rank_v, accp_v, tbuf_v, res_v,
             sh_first, sh_chosen, sh_part):
    w = lax.axis_index("s")
    iota16 = lax.iota(jnp.int32, 16)
    lane0 = iota16 == 0

    for q in range(2 * _N // 16, 2 * _NP // 16):
        seg_v[pl.ds(q * 16, 16)] = jnp.full((16,), -1.0e6, jnp.float32)
    for q in range(200 // 16, 256 // 16):
        gt_v[pl.ds(q * 16, 16)] = jnp.full((16,), 2.0e6, jnp.float32)
    for q in range(_N // 16 - 1, _NP // 16):
        conf_v[pl.ds(q * 16, 16)] = jnp.full((16,), -1.0, jnp.float32)
    pltpu.sync_copy(segf_h, seg_v.at[pl.ds(0, 2 * _N)])
    pltpu.sync_copy(gtf_h, gt_v.at[pl.ds(0, 200)])
    pltpu.sync_copy(scores_h, conf_v.at[pl.ds(0, _N)])

    for q in range(_NP // 16):
        even = iota16 * 2 + q * 32
        amin_v[pl.ds(q * 16, 16)] = plsc.load_gather(seg_v, [even])
        amax_v[pl.ds(q * 16, 16)] = plsc.load_gather(seg_v, [even + 1])
    for q in range(_LC):
        even = jnp.minimum(iota16 * 2 + q * 32, 254)
        bmin_v[pl.ds(q * 16, 16)] = plsc.load_gather(gt_v, [even])
        bmax_v[pl.ds(q * 16, 16)] = plsc.load_gather(gt_v, [even + 1])
    fix = jnp.where(iota16 >= 4, 2.0e6, bmin_v[pl.ds(96, 16)])
    bmin_v[pl.ds(96, 16)] = fix
    fix2 = jnp.where(iota16 >= 4, 2.0e6, bmax_v[pl.ds(96, 16)])
    bmax_v[pl.ds(96, 16)] = fix2

    base = w * _PT

    def phase_a_label(i2, jb):
        accs = []
        for r in range(2):
            i = i2 * 2 + r
            jidx = jnp.full((16,), jb, jnp.int32) + i
            b0 = plsc.load_gather(bmin_v, [jidx])
            b1 = plsc.load_gather(bmax_v, [jidx])
            blen = b1 - b0
            acc = jnp.full((16,), _BIGI, jnp.int32)
            for q in range(_PC):
                a0 = amin_v[pl.ds(base + q * 16, 16)]
                a1 = amax_v[pl.ds(base + q * 16, 16)]
                inter = jnp.maximum(
                    jnp.minimum(a1, b1) - jnp.maximum(a0, b0), 0.0)
                union = (a1 - a0) + blen - inter
                cand = inter >= 0.5 * union - 1.0e-3
                idx = iota16 + (base + q * 16)
                acc = jnp.minimum(acc, jnp.where(cand, idx, _BIGI))
            accs.append((i, acc))
        for i, acc in accs:
            tbuf_v[pl.ds(i * 16, 16)] = acc
        return jb

    def phase_a_chunk(jc, _):
        lax.fori_loop(0, 8, phase_a_label, jc * 16)
        res = jnp.full((16,), _BIGI, jnp.int32)
        for c in range(16):
            col = plsc.load_gather(tbuf_v, [iota16 * 16 + c])
            res = jnp.minimum(res, col)
        first_v[pl.ds(jc * 16, 16)] = res
        return 0

    lax.fori_loop(0, _LC, phase_a_chunk, 0)
    pltpu.sync_copy(first_v, sh_first.at[pl.ds(w * _LBL, _LBL)])
    plsc.subcore_barrier()

    @pl.when(w == 0)
    def _phase_b():
        for q in range(_NP // 16):
            taken_v[pl.ds(q * 16, 16)] = jnp.zeros((16,), jnp.int32)
        pltpu.sync_copy(sh_first, firstall_v)

        def red_first(j, _):
            acc = jnp.full((16,), _BIGI, jnp.int32)
            for t in range(_NT):
                acc = jnp.minimum(
                    acc, firstall_v[pl.ds(t * _LBL + j * 16, 16)])
            chosenb_v[pl.ds(j * 16, 16)] = acc
            return 0
        lax.fori_loop(0, _LC, red_first, 0)

        def phase_b_label(j, _):
            jidx = jnp.full((16,), j, jnp.int32)
            first = plsc.load_gather(chosenb_v, [jidx])[0]
            c0 = jnp.where(first < _BIGI,
                           lax.shift_right_logical(first, 4), 10 ** 6)
            b0 = plsc.load_gather(bmin_v, [jidx])
            b1 = plsc.load_gather(bmax_v, [jidx])
            blen = b1 - b0

            def cond(st):
                c, chosen = st
                return (c < _NP // 16) & (chosen >= _BIGI)

            def step(st):
                c, _ = st
                a0 = amin_v[pl.ds(c * 16, 16)]
                a1 = amax_v[pl.ds(c * 16, 16)]
                inter = jnp.maximum(
                    jnp.minimum(a1, b1) - jnp.maximum(a0, b0), 0.0)
                union = (a1 - a0) + blen - inter
                iou = inter / union
                tak = taken_v[pl.ds(c * 16, 16)]
                free = (iou > 0.5) & (tak == 0)
                fv = plsc.all_reduce_ffs(free)[0]
                ch = jnp.where(fv < 16, c * 16 + fv, jnp.int32(_BIGI))
                return c + 1, ch

            _, chosen = lax.while_loop(
                cond, step, (jnp.minimum(c0, _NP // 16), jnp.int32(_BIGI)))
            has = chosen < _BIGI
            one = jnp.full((16,), 1, jnp.int32)
            plsc.store_scatter(
                taken_v,
                [jnp.full((16,), jnp.minimum(chosen, _NP - 1), jnp.int32)],
                one, mask=lane0 & has)
            plsc.store_scatter(chosen_v, [jidx],
                               jnp.full((16,), chosen, jnp.int32), mask=lane0)
            return 0

        for q in range(_LBL // 16):
            chosen_v[pl.ds(q * 16, 16)] = jnp.full((16,), _BIGI, jnp.int32)
        lax.fori_loop(0, _M, phase_b_label, 0)
        pltpu.sync_copy(chosen_v, sh_chosen)

    plsc.subcore_barrier()

    pltpu.sync_copy(sh_chosen, chosenb_v)
    for q in range(_LBL // 16):
        idx = chosenb_v[pl.ds(q * 16, 16)]
        vmask = idx < _BIGI
        cidx = jnp.minimum(idx, _N - 1)
        cc = plsc.load_gather(conf_v, [cidx], mask=vmask)
        cc_v[pl.ds(q * 16, 16)] = jnp.where(vmask, cc, -9.0)

    def phase_c_label(i2, kb):
        accs = []
        for r in range(2):
            i = i2 * 2 + r
            kidx = jnp.full((16,), kb, jnp.int32) + i
            cvec = plsc.load_gather(cc_v, [kidx])
            mivec = plsc.load_gather(chosenb_v, [kidx])
            acc = jnp.zeros((16,), jnp.int32)
            for q in range(_PC):
                cf = conf_v[pl.ds(base + q * 16, 16)]
                gi = iota16 + (base + q * 16)
                acc = acc + jnp.where(cf > cvec, 1, 0)
                acc = acc + jnp.where((cf == cvec) & (gi < mivec), 1, 0)
            accs.append((i, acc))
        for i, acc in accs:
            tbuf_v[pl.ds(i * 16, 16)] = acc
        return kb

    def phase_c_chunk(kc, _):
        lax.fori_loop(0, 8, phase_c_label, kc * 16)
        res = jnp.zeros((16,), jnp.int32)
        for c in range(16):
            res = res + plsc.load_gather(tbuf_v, [iota16 * 16 + c])
        rpart_v[pl.ds(kc * 16, 16)] = res.astype(jnp.float32)
        return 0

    lax.fori_loop(0, _LC, phase_c_chunk, 0)
    pltpu.sync_copy(rpart_v, sh_part.at[pl.ds(w * _LBL, _LBL)])
    plsc.subcore_barrier()

    @pl.when(w == 0)
    def _phase_d():
        pltpu.sync_copy(sh_part, partall_v)

        def red_part(q, _):
            acc = jnp.zeros((16,), jnp.float32)
            for t in range(_NT):
                acc = acc + partall_v[pl.ds(t * _LBL + q * 16, 16)]
            rank_v[pl.ds(q * 16, 16)] = acc
            return 0
        lax.fori_loop(0, _LC, red_part, 0)

        def acc_loop(l4, _):
            vals = []
            for r in range(4):
                lidx = jnp.full((16,), l4 * 4, jnp.int32) + r
                r_l = plsc.load_gather(rank_v, [lidx])
                v_l = jnp.where(
                    plsc.load_gather(chosenb_v, [lidx]) < _BIGI, 1.0, 0.0)
                vals.append((r_l, v_l))
            for q in range(_LC):
                rk = rank_v[pl.ds(q * 16, 16)]
                upd = jnp.zeros((16,), jnp.float32)
                for r_l, v_l in vals:
                    upd = upd + jnp.where(r_l <= rk, v_l, 0.0)
                accp_v[pl.ds(q * 16, 16)] = accp_v[pl.ds(q * 16, 16)] + upd
            return 0

        for q in range(_LC):
            accp_v[pl.ds(q * 16, 16)] = jnp.zeros((16,), jnp.float32)
        lax.fori_loop(0, _M // 4, acc_loop, 0)

        for q in range(_LC):
            rk = rank_v[pl.ds(q * 16, 16)]
            ak = accp_v[pl.ds(q * 16, 16)]
            accp_v[pl.ds(q * 16, 16)] = ak / (rk + 1.0)

        def suff_loop(l4, suf):
            vals = []
            for r in range(4):
                lidx = jnp.full((16,), l4 * 4, jnp.int32) + r
                r_l = plsc.load_gather(rank_v, [lidx])
                p_l = plsc.load_gather(accp_v, [lidx])
                v_l = plsc.load_gather(chosenb_v, [lidx]) < _BIGI
                vals.append((r_l, p_l, v_l))
            out = []
            for q in range(_LC):
                rk = rank_v[pl.ds(q * 16, 16)]
                s = suf[q]
                for r_l, p_l, v_l in vals:
                    s = jnp.maximum(
                        s, jnp.where(v_l & (r_l >= rk), p_l, 0.0))
                out.append(s)
            return tuple(out)

        suf0 = tuple(jnp.zeros((16,), jnp.float32) for _ in range(_LC))
        suf = lax.fori_loop(0, _M // 4, suff_loop, suf0)

        apv = jnp.zeros((16,), jnp.float32)
        for q in range(_LC):
            rk = rank_v[pl.ds(q * 16, 16)]
            vk = chosenb_v[pl.ds(q * 16, 16)] < _BIGI
            apv = apv + jnp.where(vk & (rk >= 1.0), suf[q], 0.0)
        ap = _lanesum16(apv)
        res_v[...] = jnp.full((16,), ap * (1.0 / _M), jnp.float32)
        pltpu.sync_copy(res_v, out_h)


@jax.jit
def kernel(scores, segments, gt):
    segf = segments.reshape(2 * _N)
    gtf = gt.reshape(2 * _M)

    mesh = plsc.VectorSubcoreMesh(
        core_axis_name="c", subcore_axis_name="s", num_cores=1)
    f = functools.partial(
        pl.kernel, mesh=mesh,
        out_type=jax.ShapeDtypeStruct((16,), jnp.float32),
        compiler_params=pltpu.CompilerParams(needs_layout_passes=False),
        scratch_types=[
            pltpu.VMEM((2 * _NP,), jnp.float32),
            pltpu.VMEM((256,), jnp.float32),
            pltpu.VMEM((_NP,), jnp.float32),
            pltpu.VMEM((_NP,), jnp.float32),
            pltpu.VMEM((_NP,), jnp.float32),
            pltpu.VMEM((_LBL,), jnp.float32),
            pltpu.VMEM((_LBL,), jnp.float32),
            pltpu.VMEM((_LBL,), jnp.int32),
            pltpu.VMEM((_NT * _LBL,), jnp.int32),
            pltpu.VMEM((_NP,), jnp.int32),
            pltpu.VMEM((_LBL,), jnp.int32),
            pltpu.VMEM((_LBL,), jnp.int32),
            pltpu.VMEM((_LBL,), jnp.float32),
            pltpu.VMEM((_LBL,), jnp.float32),
            pltpu.VMEM((_NT * _LBL,), jnp.float32),
            pltpu.VMEM((_LBL,), jnp.float32),
            pltpu.VMEM((_LBL,), jnp.float32),
            pltpu.VMEM((256,), jnp.int32),
            pltpu.VMEM((16,), jnp.float32),
            pltpu.VMEM_SHARED((_NT * _LBL,), jnp.int32),
            pltpu.VMEM_SHARED((_LBL,), jnp.int32),
            pltpu.VMEM_SHARED((_NT * _LBL,), jnp.float32),
        ])(_sc_body)
    out = f(scores, segf, gtf)
    return out[0]


# speedup vs baseline: 1.2323x; 1.1157x over previous
import functools

import jax
import jax.numpy as jnp
from jax import lax
from jax.experimental import pallas as pl
from jax.experimental.pallas import tpu as pltpu
from jax.experimental.pallas import tpu_sc as plsc

_N = 5000
_M = 100
_NP = 5120
_NT = 16
_PT = _NP // _NT
_PC = _PT // 16
_BIGI = 1 << 30
_LBL = 128
_LC = 7


def _lanesum16(x):
    s = x[0]
    for i in range(1, 16):
        s = s + x[i]
    return s


def _sc_body(amin_h, amax_h, conf_h, bmin_h, bmax_h, out_h,
             amin_v, amax_v, conf_v, bmin_v, bmax_v,
             first_v, firstall_v, taken_v, chosen_v, chosenb_v, cc_v,
             rpart_v, partall_v, rank_v, accp_v, tbuf_v, res_v,
             sh_first, sh_chosen, sh_part):
    w = lax.axis_index("s")
    iota16 = lax.iota(jnp.int32, 16)
    lane0 = iota16 == 0

    pltpu.sync_copy(amin_h, amin_v)
    pltpu.sync_copy(amax_h, amax_v)
    pltpu.sync_copy(conf_h, conf_v)
    pltpu.sync_copy(bmin_h, bmin_v)
    pltpu.sync_copy(bmax_h, bmax_v)

    base = w * _PT

    def phase_a_label(i2, jb):
        accs = []
        for r in range(2):
            i = i2 * 2 + r
            jidx = jnp.full((16,), jb, jnp.int32) + i
            b0 = plsc.load_gather(bmin_v, [jidx])
            b1 = plsc.load_gather(bmax_v, [jidx])
            blen = b1 - b0
            acc = jnp.full((16,), _BIGI, jnp.int32)
            for q in range(_PC):
                a0 = amin_v[pl.ds(base + q * 16, 16)]
                a1 = amax_v[pl.ds(base + q * 16, 16)]
                inter = jnp.maximum(
                    jnp.minimum(a1, b1) - jnp.maximum(a0, b0), 0.0)
                union = (a1 - a0) + blen - inter
                cand = inter >= 0.5 * union - 1.0e-3
                idx = iota16 + (base + q * 16)
                acc = jnp.minimum(acc, jnp.where(cand, idx, _BIGI))
            accs.append((i, acc))
        for i, acc in accs:
            tbuf_v[pl.ds(i * 16, 16)] = acc
        return jb

    def phase_a_chunk(jc, _):
        lax.fori_loop(0, 8, phase_a_label, jc * 16)
        res = jnp.full((16,), _BIGI, jnp.int32)
        for c in range(16):
            col = plsc.load_gather(tbuf_v, [iota16 * 16 + c])
            res = jnp.minimum(res, col)
        first_v[pl.ds(jc * 16, 16)] = res
        return 0

    lax.fori_loop(0, _LC, phase_a_chunk, 0)
    pltpu.sync_copy(first_v, sh_first.at[pl.ds(w * _LBL, _LBL)])
    plsc.subcore_barrier()

    @pl.when(w == 0)
    def _phase_b():
        for q in range(_NP // 16):
            taken_v[pl.ds(q * 16, 16)] = jnp.zeros((16,), jnp.int32)
        pltpu.sync_copy(sh_first, firstall_v)

        def red_first(j, _):
            acc = jnp.full((16,), _BIGI, jnp.int32)
            for t in range(_NT):
                acc = jnp.minimum(
                    acc, firstall_v[pl.ds(t * _LBL + j * 16, 16)])
            chosenb_v[pl.ds(j * 16, 16)] = acc
            return 0
        lax.fori_loop(0, _LC, red_first, 0)

        def phase_b_label(j, _):
            jidx = jnp.full((16,), j, jnp.int32)
            first = plsc.load_gather(chosenb_v, [jidx])[0]
            c0 = jnp.where(first < _BIGI,
                           lax.shift_right_logical(first, 4), 10 ** 6)
            b0 = plsc.load_gather(bmin_v, [jidx])
            b1 = plsc.load_gather(bmax_v, [jidx])
            blen = b1 - b0

            def cond(st):
                c, chosen = st
                return (c < _NP // 16) & (chosen >= _BIGI)

            def step(st):
                c, _ = st
                a0 = amin_v[pl.ds(c * 16, 16)]
                a1 = amax_v[pl.ds(c * 16, 16)]
                inter = jnp.maximum(
                    jnp.minimum(a1, b1) - jnp.maximum(a0, b0), 0.0)
                union = (a1 - a0) + blen - inter
                iou = inter / union
                tak = taken_v[pl.ds(c * 16, 16)]
                free = (iou > 0.5) & (tak == 0)
                fv = plsc.all_reduce_ffs(free)[0]
                ch = jnp.where(fv < 16, c * 16 + fv, jnp.int32(_BIGI))
                return c + 1, ch

            _, chosen = lax.while_loop(
                cond, step, (jnp.minimum(c0, _NP // 16), jnp.int32(_BIGI)))
            has = chosen < _BIGI
            one = jnp.full((16,), 1, jnp.int32)
            plsc.store_scatter(
                taken_v,
                [jnp.full((16,), jnp.minimum(chosen, _NP - 1), jnp.int32)],
                one, mask=lane0 & has)
            plsc.store_scatter(chosen_v, [jidx],
                               jnp.full((16,), chosen, jnp.int32), mask=lane0)
            return 0

        for q in range(_LBL // 16):
            chosen_v[pl.ds(q * 16, 16)] = jnp.full((16,), _BIGI, jnp.int32)
        lax.fori_loop(0, _M, phase_b_label, 0)
        pltpu.sync_copy(chosen_v, sh_chosen)

    plsc.subcore_barrier()

    pltpu.sync_copy(sh_chosen, chosenb_v)
    for q in range(_LBL // 16):
        idx = chosenb_v[pl.ds(q * 16, 16)]
        vmask = idx < _BIGI
        cidx = jnp.minimum(idx, _N - 1)
        cc = plsc.load_gather(conf_v, [cidx], mask=vmask)
        cc_v[pl.ds(q * 16, 16)] = jnp.where(vmask, cc, -9.0)

    def phase_c_label(i2, kb):
        accs = []
        for r in range(2):
            i = i2 * 2 + r
            kidx = jnp.full((16,), kb, jnp.int32) + i
            cvec = plsc.load_gather(cc_v, [kidx])
            mivec = plsc.load_gather(chosenb_v, [kidx])
            acc = jnp.zeros((16,), jnp.int32)
            for q in range(_PC):
                cf = conf_v[pl.ds(base + q * 16, 16)]
                gi = iota16 + (base + q * 16)
                acc = acc + jnp.where(cf > cvec, 1, 0)
                acc = acc + jnp.where((cf == cvec) & (gi < mivec), 1, 0)
            accs.append((i, acc))
        for i, acc in accs:
            tbuf_v[pl.ds(i * 16, 16)] = acc
        return kb

    def phase_c_chunk(kc, _):
        lax.fori_loop(0, 8, phase_c_label, kc * 16)
        res = jnp.zeros((16,), jnp.int32)
        for c in range(16):
            res = res + plsc.load_gather(tbuf_v, [iota16 * 16 + c])
        rpart_v[pl.ds(kc * 16, 16)] = res.astype(jnp.float32)
        return 0

    lax.fori_loop(0, _LC, phase_c_chunk, 0)
    pltpu.sync_copy(rpart_v, sh_part.at[pl.ds(w * _LBL, _LBL)])
    plsc.subcore_barrier()

    @pl.when(w == 0)
    def _phase_d():
        pltpu.sync_copy(sh_part, partall_v)

        def red_part(q, _):
            acc = jnp.zeros((16,), jnp.float32)
            for t in range(_NT):
                acc = acc + partall_v[pl.ds(t * _LBL + q * 16, 16)]
            rank_v[pl.ds(q * 16, 16)] = acc
            return 0
        lax.fori_loop(0, _LC, red_part, 0)

        def acc_loop(l4, _):
            vals = []
            for r in range(4):
                lidx = jnp.full((16,), l4 * 4, jnp.int32) + r
                r_l = plsc.load_gather(rank_v, [lidx])
                v_l = jnp.where(
                    plsc.load_gather(chosenb_v, [lidx]) < _BIGI, 1.0, 0.0)
                vals.append((r_l, v_l))
            for q in range(_LC):
                rk = rank_v[pl.ds(q * 16, 16)]
                upd = jnp.zeros((16,), jnp.float32)
                for r_l, v_l in vals:
                    upd = upd + jnp.where(r_l <= rk, v_l, 0.0)
                accp_v[pl.ds(q * 16, 16)] = accp_v[pl.ds(q * 16, 16)] + upd
            return 0

        for q in range(_LC):
            accp_v[pl.ds(q * 16, 16)] = jnp.zeros((16,), jnp.float32)
        lax.fori_loop(0, _M // 4, acc_loop, 0)

        for q in range(_LC):
            rk = rank_v[pl.ds(q * 16, 16)]
            ak = accp_v[pl.ds(q * 16, 16)]
            accp_v[pl.ds(q * 16, 16)] = ak / (rk + 1.0)

        def suff_loop(l4, suf):
            vals = []
            for r in range(4):
                lidx = jnp.full((16,), l4 * 4, jnp.int32) + r
                r_l = plsc.load_gather(rank_v, [lidx])
                p_l = plsc.load_gather(accp_v, [lidx])
                v_l = plsc.load_gather(chosenb_v, [lidx]) < _BIGI
                vals.append((r_l, p_l, v_l))
            out = []
            for q in range(_LC):
                rk = rank_v[pl.ds(q * 16, 16)]
                s = suf[q]
                for r_l, p_l, v_l in vals:
                    s = jnp.maximum(
                        s, jnp.where(v_l & (r_l >= rk), p_l, 0.0))
                out.append(s)
            return tuple(out)

        suf0 = tuple(jnp.zeros((16,), jnp.float32) for _ in range(_LC))
        suf = lax.fori_loop(0, _M // 4, suff_loop, suf0)

        apv = jnp.zeros((16,), jnp.float32)
        for q in range(_LC):
            rk = rank_v[pl.ds(q * 16, 16)]
            vk = chosenb_v[pl.ds(q * 16, 16)] < _BIGI
            apv = apv + jnp.where(vk & (rk >= 1.0), suf[q], 0.0)
        ap = _lanesum16(apv)
        res_v[...] = jnp.full((16,), ap * (1.0 / _M), jnp.float32)
        pltpu.sync_copy(res_v, out_h)


@jax.jit
def kernel(scores, segments, gt):
    pad = _NP - _N
    amin = jnp.pad(segments[:, 0], (0, pad), constant_values=-1.0e6)
    amax = jnp.pad(segments[:, 1], (0, pad), constant_values=-1.0e6)
    conf = jnp.pad(scores, (0, pad), constant_values=-1.0)
    bmin = jnp.pad(gt[:, 0], (0, _LBL - _M), constant_values=2.0e6)
    bmax = jnp.pad(gt[:, 1], (0, _LBL - _M), constant_values=2.0e6)

    mesh = plsc.VectorSubcoreMesh(
        core_axis_name="c", subcore_axis_name="s", num_cores=1)
    f = functools.partial(
        pl.kernel, mesh=mesh,
        out_type=jax.ShapeDtypeStruct((16,), jnp.float32),
        compiler_params=pltpu.CompilerParams(needs_layout_passes=False),
        scratch_types=[
            pltpu.VMEM((_NP,), jnp.float32),
            pltpu.VMEM((_NP,), jnp.float32),
            pltpu.VMEM((_NP,), jnp.float32),
            pltpu.VMEM((_LBL,), jnp.float32),
            pltpu.VMEM((_LBL,), jnp.float32),
            pltpu.VMEM((_LBL,), jnp.int32),
            pltpu.VMEM((_NT * _LBL,), jnp.int32),
            pltpu.VMEM((_NP,), jnp.int32),
            pltpu.VMEM((_LBL,), jnp.int32),
            pltpu.VMEM((_LBL,), jnp.int32),
            pltpu.VMEM((_LBL,), jnp.float32),
            pltpu.VMEM((_LBL,), jnp.float32),
            pltpu.VMEM((_NT * _LBL,), jnp.float32),
            pltpu.VMEM((_LBL,), jnp.float32),
            pltpu.VMEM((_LBL,), jnp.float32),
            pltpu.VMEM((256,), jnp.int32),
            pltpu.VMEM((16,), jnp.float32),
            pltpu.VMEM_SHARED((_NT * _LBL,), jnp.int32),
            pltpu.VMEM_SHARED((_LBL,), jnp.int32),
            pltpu.VMEM_SHARED((_NT * _LBL,), jnp.float32),
        ])(_sc_body)
    out = f(amin, amax, conf, bmin, bmax)
    return out[0]


# speedup vs baseline: 1.2790x; 1.0379x over previous
import functools

import jax
import jax.numpy as jnp
from jax import lax
from jax.experimental import pallas as pl
from jax.experimental.pallas import tpu as pltpu
from jax.experimental.pallas import tpu_sc as plsc

_N = 5000
_M = 100
_NP = 5120
_NT = 16
_PT = _NP // _NT
_PC = _PT // 16
_BIGI = 1 << 30
_LBL = 128
_LC = 7


def _lanesum16(x):
    s = x[0]
    for i in range(1, 16):
        s = s + x[i]
    return s


def _sc_body(amin_h, amax_h, conf_h, bmin_h, bmax_h, out_h,
             amin_v, amax_v, conf_v, bmin_v, bmax_v,
             first_v, firstall_v, taken_v, chosen_v, chosenb_v, cc_v,
             rpart_v, partall_v, rank_v, accp_v, tbuf_v, res_v, dsem,
             sh_first, sh_chosen, sh_part):
    w = lax.axis_index("s")
    iota16 = lax.iota(jnp.int32, 16)
    lane0 = iota16 == 0

    copies = [pltpu.async_copy(amin_h, amin_v, dsem),
              pltpu.async_copy(amax_h, amax_v, dsem),
              pltpu.async_copy(conf_h, conf_v, dsem),
              pltpu.async_copy(bmin_h, bmin_v, dsem),
              pltpu.async_copy(bmax_h, bmax_v, dsem)]
    for cp in copies:
        cp.wait()

    base = w * _PT

    def phase_a_label(i2, jb):
        accs = []
        for r in range(2):
            i = i2 * 2 + r
            jidx = jnp.full((16,), jb, jnp.int32) + i
            b0 = plsc.load_gather(bmin_v, [jidx])
            b1 = plsc.load_gather(bmax_v, [jidx])
            blen = b1 - b0
            acc = jnp.full((16,), _BIGI, jnp.int32)
            for q in range(_PC):
                a0 = amin_v[pl.ds(base + q * 16, 16)]
                a1 = amax_v[pl.ds(base + q * 16, 16)]
                inter = jnp.maximum(
                    jnp.minimum(a1, b1) - jnp.maximum(a0, b0), 0.0)
                union = (a1 - a0) + blen - inter
                cand = inter >= 0.5 * union - 1.0e-3
                idx = iota16 + (base + q * 16)
                acc = jnp.minimum(acc, jnp.where(cand, idx, _BIGI))
            accs.append((i, acc))
        for i, acc in accs:
            tbuf_v[pl.ds(i * 16, 16)] = acc
        return jb

    def phase_a_chunk(jc, _):
        lax.fori_loop(0, 8, phase_a_label, jc * 16)
        res = jnp.full((16,), _BIGI, jnp.int32)
        for c in range(16):
            col = plsc.load_gather(tbuf_v, [iota16 * 16 + c])
            res = jnp.minimum(res, col)
        first_v[pl.ds(jc * 16, 16)] = res
        return 0

    lax.fori_loop(0, _LC, phase_a_chunk, 0)
    pltpu.sync_copy(first_v, sh_first.at[pl.ds(w * _LBL, _LBL)])
    plsc.subcore_barrier()

    @pl.when(w == 0)
    def _phase_b():
        for q in range(_NP // 16):
            taken_v[pl.ds(q * 16, 16)] = jnp.zeros((16,), jnp.int32)
        pltpu.sync_copy(sh_first, firstall_v)

        def red_first(j, _):
            acc = jnp.full((16,), _BIGI, jnp.int32)
            for t in range(_NT):
                acc = jnp.minimum(
                    acc, firstall_v[pl.ds(t * _LBL + j * 16, 16)])
            chosenb_v[pl.ds(j * 16, 16)] = acc
            return 0
        lax.fori_loop(0, _LC, red_first, 0)

        def phase_b_label(j, _):
            jidx = jnp.full((16,), j, jnp.int32)
            first = plsc.load_gather(chosenb_v, [jidx])[0]
            c0 = jnp.where(first < _BIGI,
                           lax.shift_right_logical(first, 4), 10 ** 6)
            b0 = plsc.load_gather(bmin_v, [jidx])
            b1 = plsc.load_gather(bmax_v, [jidx])
            blen = b1 - b0

            def cond(st):
                c, chosen = st
                return (c < _NP // 16) & (chosen >= _BIGI)

            def step(st):
                c, _ = st
                a0 = amin_v[pl.ds(c * 16, 16)]
                a1 = amax_v[pl.ds(c * 16, 16)]
                inter = jnp.maximum(
                    jnp.minimum(a1, b1) - jnp.maximum(a0, b0), 0.0)
                union = (a1 - a0) + blen - inter
                iou = inter / union
                tak = taken_v[pl.ds(c * 16, 16)]
                free = (iou > 0.5) & (tak == 0)
                fv = plsc.all_reduce_ffs(free)[0]
                ch = jnp.where(fv < 16, c * 16 + fv, jnp.int32(_BIGI))
                return c + 1, ch

            _, chosen = lax.while_loop(
                cond, step, (jnp.minimum(c0, _NP // 16), jnp.int32(_BIGI)))
            has = chosen < _BIGI
            one = jnp.full((16,), 1, jnp.int32)
            plsc.store_scatter(
                taken_v,
                [jnp.full((16,), jnp.minimum(chosen, _NP - 1), jnp.int32)],
                one, mask=lane0 & has)
            plsc.store_scatter(chosen_v, [jidx],
                               jnp.full((16,), chosen, jnp.int32), mask=lane0)
            return 0

        for q in range(_LBL // 16):
            chosen_v[pl.ds(q * 16, 16)] = jnp.full((16,), _BIGI, jnp.int32)
        lax.fori_loop(0, _M, phase_b_label, 0)
        pltpu.sync_copy(chosen_v, sh_chosen)

    plsc.subcore_barrier()

    pltpu.sync_copy(sh_chosen, chosenb_v)
    for q in range(_LBL // 16):
        idx = chosenb_v[pl.ds(q * 16, 16)]
        vmask = idx < _BIGI
        cidx = jnp.minimum(idx, _N - 1)
        cc = plsc.load_gather(conf_v, [cidx], mask=vmask)
        cc_v[pl.ds(q * 16, 16)] = jnp.where(vmask, cc, -9.0)

    def phase_c_label(i2, kb):
        accs = []
        for r in range(2):
            i = i2 * 2 + r
            kidx = jnp.full((16,), kb, jnp.int32) + i
            cvec = plsc.load_gather(cc_v, [kidx])
            mivec = plsc.load_gather(chosenb_v, [kidx])
            acc = jnp.zeros((16,), jnp.int32)
            for q in range(_PC):
                cf = conf_v[pl.ds(base + q * 16, 16)]
                gi = iota16 + (base + q * 16)
                acc = acc + jnp.where(cf > cvec, 1, 0)
                acc = acc + jnp.where((cf == cvec) & (gi < mivec), 1, 0)
            accs.append((i, acc))
        for i, acc in accs:
            tbuf_v[pl.ds(i * 16, 16)] = acc
        return kb

    def phase_c_chunk(kc, _):
        lax.fori_loop(0, 8, phase_c_label, kc * 16)
        res = jnp.zeros((16,), jnp.int32)
        for c in range(16):
            res = res + plsc.load_gather(tbuf_v, [iota16 * 16 + c])
        rpart_v[pl.ds(kc * 16, 16)] = res.astype(jnp.float32)
        return 0

    lax.fori_loop(0, _LC, phase_c_chunk, 0)
    pltpu.sync_copy(rpart_v, sh_part.at[pl.ds(w * _LBL, _LBL)])
    plsc.subcore_barrier()

    @pl.when(w == 0)
    def _phase_d():
        pltpu.sync_copy(sh_part, partall_v)

        def red_part(q, _):
            acc = jnp.zeros((16,), jnp.float32)
            for t in range(_NT):
                acc = acc + partall_v[pl.ds(t * _LBL + q * 16, 16)]
            rank_v[pl.ds(q * 16, 16)] = acc
            return 0
        lax.fori_loop(0, _LC, red_part, 0)

        def acc_loop(l4, _):
            vals = []
            for r in range(4):
                lidx = jnp.full((16,), l4 * 4, jnp.int32) + r
                r_l = plsc.load_gather(rank_v, [lidx])
                v_l = jnp.where(
                    plsc.load_gather(chosenb_v, [lidx]) < _BIGI, 1.0, 0.0)
                vals.append((r_l, v_l))
            for q in range(_LC):
                rk = rank_v[pl.ds(q * 16, 16)]
                upd = jnp.zeros((16,), jnp.float32)
                for r_l, v_l in vals:
                    upd = upd + jnp.where(r_l <= rk, v_l, 0.0)
                accp_v[pl.ds(q * 16, 16)] = accp_v[pl.ds(q * 16, 16)] + upd
            return 0

        for q in range(_LC):
            accp_v[pl.ds(q * 16, 16)] = jnp.zeros((16,), jnp.float32)
        lax.fori_loop(0, _M // 4, acc_loop, 0)

        for q in range(_LC):
            rk = rank_v[pl.ds(q * 16, 16)]
            ak = accp_v[pl.ds(q * 16, 16)]
            accp_v[pl.ds(q * 16, 16)] = ak / (rk + 1.0)

        def suff_loop(l4, suf):
            vals = []
            for r in range(4):
                lidx = jnp.full((16,), l4 * 4, jnp.int32) + r
                r_l = plsc.load_gather(rank_v, [lidx])
                p_l = plsc.load_gather(accp_v, [lidx])
                v_l = plsc.load_gather(chosenb_v, [lidx]) < _BIGI
                vals.append((r_l, p_l, v_l))
            out = []
            for q in range(_LC):
                rk = rank_v[pl.ds(q * 16, 16)]
                s = suf[q]
                for r_l, p_l, v_l in vals:
                    s = jnp.maximum(
                        s, jnp.where(v_l & (r_l >= rk), p_l, 0.0))
                out.append(s)
            return tuple(out)

        suf0 = tuple(jnp.zeros((16,), jnp.float32) for _ in range(_LC))
        suf = lax.fori_loop(0, _M // 4, suff_loop, suf0)

        apv = jnp.zeros((16,), jnp.float32)
        for q in range(_LC):
            rk = rank_v[pl.ds(q * 16, 16)]
            vk = chosenb_v[pl.ds(q * 16, 16)] < _BIGI
            apv = apv + jnp.where(vk & (rk >= 1.0), suf[q], 0.0)
        ap = _lanesum16(apv)
        res_v[...] = jnp.full((16,), ap * (1.0 / _M), jnp.float32)
        pltpu.sync_copy(res_v, out_h)


@jax.jit
def kernel(scores, segments, gt):
    pad = _NP - _N
    amin = jnp.pad(segments[:, 0], (0, pad), constant_values=-1.0e6)
    amax = jnp.pad(segments[:, 1], (0, pad), constant_values=-1.0e6)
    conf = jnp.pad(scores, (0, pad), constant_values=-1.0)
    bmin = jnp.pad(gt[:, 0], (0, _LBL - _M), constant_values=2.0e6)
    bmax = jnp.pad(gt[:, 1], (0, _LBL - _M), constant_values=2.0e6)

    mesh = plsc.VectorSubcoreMesh(
        core_axis_name="c", subcore_axis_name="s", num_cores=1)
    f = functools.partial(
        pl.kernel, mesh=mesh,
        out_type=jax.ShapeDtypeStruct((16,), jnp.float32),
        compiler_params=pltpu.CompilerParams(needs_layout_passes=False),
        scratch_types=[
            pltpu.VMEM((_NP,), jnp.float32),
            pltpu.VMEM((_NP,), jnp.float32),
            pltpu.VMEM((_NP,), jnp.float32),
            pltpu.VMEM((_LBL,), jnp.float32),
            pltpu.VMEM((_LBL,), jnp.float32),
            pltpu.VMEM((_LBL,), jnp.int32),
            pltpu.VMEM((_NT * _LBL,), jnp.int32),
            pltpu.VMEM((_NP,), jnp.int32),
            pltpu.VMEM((_LBL,), jnp.int32),
            pltpu.VMEM((_LBL,), jnp.int32),
            pltpu.VMEM((_LBL,), jnp.float32),
            pltpu.VMEM((_LBL,), jnp.float32),
            pltpu.VMEM((_NT * _LBL,), jnp.float32),
            pltpu.VMEM((_LBL,), jnp.float32),
            pltpu.VMEM((_LBL,), jnp.float32),
            pltpu.VMEM((256,), jnp.int32),
            pltpu.VMEM((16,), jnp.float32),
            pltpu.SemaphoreType.DMA,
            pltpu.VMEM_SHARED((_NT * _LBL,), jnp.int32),
            pltpu.VMEM_SHARED((_LBL,), jnp.int32),
            pltpu.VMEM_SHARED((_NT * _LBL,), jnp.float32),
        ])(_sc_body)
    out = f(amin, amax, conf, bmin, bmax)
    return out[0]
